# baseline (device time: 92558 ns/iter reference)
import numpy as np
import jax
import jax.numpy as jnp
from jax import lax
from jax.experimental import pallas as pl
from jax.experimental.pallas import tpu as pltpu

N_DEV = 16
SQ = 1024
D = 1024
HQ = 8
DH = 128
HD = HQ * DH
HALF = SQ // 2
SCALE = 0.08838834764831843
SIZES = (512, 256, 128, 64)


def _rope_tables():
    inv = 1.0 / (10000.0 ** (np.arange(0, DH, 2) / DH))
    pos = np.arange(SQ)[:, None] * inv[None, :]
    cos = np.repeat(np.cos(pos), 2, axis=-1).astype(np.float32)
    sin = np.repeat(np.sin(pos), 2, axis=-1).astype(np.float32)
    return cos, sin


_COS, _SIN = _rope_tables()


def kernel(x, Wq, Wk, Wv, Wo):
    x2 = x.reshape(SQ, D)
    cos = jnp.asarray(_COS)
    sin = jnp.asarray(_SIN)

    def body(x_ref, wq_ref, wk_ref, wv_ref, wo_ref, cos_ref, sin_ref,
             out_ref, q_ref, k_ref, v_ref, ctx_ref, part_ref,
             sbuf1, sbuf2, sbuf3, sbuf4,
             rbuf1, rbuf2, rbuf3, rbuf4,
             rs_send, rs_recv, ag_send, ag_recv):
        my = lax.axis_index("i")
        z = my // 4
        r = my % 4
        y = jnp.where(r >= 2, 1, 0)
        xc = jnp.where((r == 1) | (r == 2), 1, 0)
        z0 = z % 2
        z1 = z // 2

        def lid(xx, yy, zz):
            return 4 * zz + 3 * yy + xx * (1 - 2 * yy)

        partners = [
            lid(xc, 1 - y, z),
            lid(1 - xc, y, z),
            lid(xc, y, z + 1 - 2 * z0),
            lid(xc, y, z + 2 - 4 * z1),
        ]
        keep = [y * 512]
        send = [(1 - y) * 512]
        keep.append(keep[0] + xc * 256)
        send.append(keep[0] + (1 - xc) * 256)
        keep.append(keep[1] + z0 * 128)
        send.append(keep[1] + (1 - z0) * 128)
        keep.append(keep[2] + z1 * 64)
        send.append(keep[2] + (1 - z1) * 64)

        barrier = pltpu.get_barrier_semaphore()
        for p in partners:
            pl.semaphore_signal(barrier, inc=1, device_id=(p,),
                                device_id_type=pl.DeviceIdType.MESH)
        pl.semaphore_wait(barrier, 4)

        xv = x_ref[:, :]
        cosv = jnp.concatenate([cos_ref[:, :]] * HQ, axis=1)
        sinv = jnp.concatenate([sin_ref[:, :]] * HQ, axis=1)
        col = lax.broadcasted_iota(jnp.int32, (SQ, HD), 1)
        even = (col % 2) == 0

        def rope(t):
            t_next = pltpu.roll(t, HD - 1, 1)
            t_prev = pltpu.roll(t, 1, 1)
            t_r = jnp.where(even, -t_next, t_prev)
            return t * cosv + t_r * sinv

        xvb = xv.astype(jnp.bfloat16)
        q_ref[:, :] = rope(jnp.dot(xvb, wq_ref[:, :].astype(jnp.bfloat16),
                                   preferred_element_type=jnp.float32))
        k_ref[:, :] = rope(jnp.dot(xvb, wk_ref[:, :].astype(jnp.bfloat16),
                                   preferred_element_type=jnp.float32))
        v_ref[:, :] = jnp.dot(xvb, wv_ref[:, :].astype(jnp.bfloat16),
                              preferred_element_type=jnp.float32
                              ).astype(jnp.bfloat16)

        def attn_part(base):
            for h in range(HQ):
                sl = slice(h * DH, (h + 1) * DH)
                s = lax.dot_general(
                    q_ref[pl.ds(base, HALF), sl], k_ref[:, sl],
                    (((1,), (1,)), ((), ())),
                    preferred_element_type=jnp.float32) * SCALE
                m = jnp.max(s, axis=1, keepdims=True)
                w = jnp.exp(s - m)
                w = w / jnp.sum(w, axis=1, keepdims=True)
                ctx_ref[:, sl] = jnp.dot(
                    w.astype(jnp.bfloat16), v_ref[:, sl],
                    preferred_element_type=jnp.float32).astype(jnp.bfloat16)
            part_ref[pl.ds(base, HALF), :] = jnp.dot(
                ctx_ref[:, :], wo_ref[:, :].astype(jnp.bfloat16),
                preferred_element_type=jnp.float32)

        rbufs = [rbuf1, rbuf2, rbuf3, rbuf4]
        sbufs = [sbuf1, sbuf2, sbuf3, sbuf4]

        def rs_step(s):
            sbufs[s][:, :] = part_ref[pl.ds(send[s], SIZES[s]), :].astype(
                jnp.bfloat16)
            return pltpu.make_async_remote_copy(
                src_ref=sbufs[s],
                dst_ref=rbufs[s],
                send_sem=rs_send.at[s],
                recv_sem=rs_recv.at[s],
                device_id=(partners[s],),
                device_id_type=pl.DeviceIdType.MESH,
            )

        attn_part(send[0])
        rdma1 = rs_step(0)
        rdma1.start()
        attn_part(keep[0])
        rdma1.wait()
        part_ref[pl.ds(keep[0], SIZES[0]), :] = (
            part_ref[pl.ds(keep[0], SIZES[0]), :]
            + rbuf1[:, :].astype(jnp.float32))

        for s in (1, 2, 3):
            rdma = rs_step(s)
            rdma.start()
            rdma.wait()
            part_ref[pl.ds(keep[s], SIZES[s]), :] = (
                part_ref[pl.ds(keep[s], SIZES[s]), :]
                + rbufs[s][:, :].astype(jnp.float32))

        out_ref[pl.ds(keep[3], 64), :] = part_ref[pl.ds(keep[3], 64),
                                                  :].astype(jnp.bfloat16)

        for s in (3, 2, 1, 0):
            rdma = pltpu.make_async_remote_copy(
                src_ref=out_ref.at[pl.ds(keep[s], SIZES[s]), :],
                dst_ref=out_ref.at[pl.ds(keep[s], SIZES[s]), :],
                send_sem=ag_send.at[s],
                recv_sem=ag_recv.at[s],
                device_id=(partners[s],),
                device_id_type=pl.DeviceIdType.MESH,
            )
            rdma.start()
            rdma.wait()

    out = pl.pallas_call(
        body,
        out_shape=jax.ShapeDtypeStruct((SQ, D), jnp.bfloat16),
        in_specs=[pl.BlockSpec(memory_space=pltpu.VMEM)] * 7,
        out_specs=pl.BlockSpec(memory_space=pltpu.VMEM),
        scratch_shapes=[
            pltpu.VMEM((SQ, HD), jnp.float32),
            pltpu.VMEM((SQ, HD), jnp.float32),
            pltpu.VMEM((SQ, HD), jnp.bfloat16),
            pltpu.VMEM((HALF, HD), jnp.bfloat16),
            pltpu.VMEM((SQ, D), jnp.float32),
            pltpu.VMEM((512, D), jnp.bfloat16),
            pltpu.VMEM((256, D), jnp.bfloat16),
            pltpu.VMEM((128, D), jnp.bfloat16),
            pltpu.VMEM((64, D), jnp.bfloat16),
            pltpu.VMEM((512, D), jnp.bfloat16),
            pltpu.VMEM((256, D), jnp.bfloat16),
            pltpu.VMEM((128, D), jnp.bfloat16),
            pltpu.VMEM((64, D), jnp.bfloat16),
            pltpu.SemaphoreType.DMA((4,)),
            pltpu.SemaphoreType.DMA((4,)),
            pltpu.SemaphoreType.DMA((4,)),
            pltpu.SemaphoreType.DMA((4,)),
        ],
        compiler_params=pltpu.CompilerParams(
            collective_id=0,
            vmem_limit_bytes=128 * 1024 * 1024,
        ),
    )(x2, Wq, Wk, Wv, Wo, cos, sin)
    return out.astype(jnp.float32).reshape(1, SQ, D)


# device time: 80027 ns/iter; 1.1566x vs baseline; 1.1566x over previous
import os
import numpy as np
import jax
import jax.numpy as jnp
from jax import lax
from jax.experimental import pallas as pl
from jax.experimental.pallas import tpu as pltpu

N_DEV = 16
SQ = 1024
D = 1024
HQ = 8
DH = 128
HD = HQ * DH
HALF = SQ // 2
SCALE = 0.08838834764831843
SIZES = (512, 256, 128, 64)


def _rope_tables():
    inv = 1.0 / (10000.0 ** (np.arange(0, DH, 2) / DH))
    pos = np.arange(SQ)[:, None] * inv[None, :]
    cos = np.repeat(np.cos(pos), 2, axis=-1).astype(np.float32)
    sin = np.repeat(np.sin(pos), 2, axis=-1).astype(np.float32)
    return cos, sin


_COS, _SIN = _rope_tables()
_ABLATE = os.environ.get("KERNEL_ABLATE", "")


def kernel(x, Wq, Wk, Wv, Wo):
    x2 = x.reshape(SQ, D)
    cos = jnp.asarray(_COS)
    sin = jnp.asarray(_SIN)

    def body(x_ref, wq_ref, wk_ref, wv_ref, wo_ref, cos_ref, sin_ref,
             out_ref, q_ref, k_ref, v_ref, ctx_ref, part_ref,
             sbuf1, sbuf2, sbuf3, sbuf4,
             rbuf1, rbuf2, rbuf3, rbuf4,
             rs_send, rs_recv, ag_send, ag_recv):
        my = lax.axis_index("i")
        z = my // 4
        r = my % 4
        y = jnp.where(r >= 2, 1, 0)
        xc = jnp.where((r == 1) | (r == 2), 1, 0)
        z0 = z % 2
        z1 = z // 2

        def lid(xx, yy, zz):
            return 4 * zz + 3 * yy + xx * (1 - 2 * yy)

        partners = [
            lid(xc, 1 - y, z),
            lid(1 - xc, y, z),
            lid(xc, y, z + 1 - 2 * z0),
            lid(xc, y, z + 2 - 4 * z1),
        ]
        keep = [y * 512]
        send = [(1 - y) * 512]
        keep.append(keep[0] + xc * 256)
        send.append(keep[0] + (1 - xc) * 256)
        keep.append(keep[1] + z0 * 128)
        send.append(keep[1] + (1 - z0) * 128)
        keep.append(keep[2] + z1 * 64)
        send.append(keep[2] + (1 - z1) * 64)

        if _ABLATE != "nocomm":
            barrier = pltpu.get_barrier_semaphore()
            for p in partners:
                pl.semaphore_signal(barrier, inc=1, device_id=(p,),
                                    device_id_type=pl.DeviceIdType.MESH)
            pl.semaphore_wait(barrier, 4)

        xv = x_ref[:, :]
        cosv = jnp.concatenate([cos_ref[:, :]] * HQ, axis=1)
        sinv = jnp.concatenate([sin_ref[:, :]] * HQ, axis=1)
        col = lax.broadcasted_iota(jnp.int32, (SQ, HD), 1)
        even = (col % 2) == 0

        def rope(t):
            t_next = pltpu.roll(t, HD - 1, 1)
            t_prev = pltpu.roll(t, 1, 1)
            t_r = jnp.where(even, -t_next, t_prev)
            return t * cosv + t_r * sinv

        xvb = xv.astype(jnp.bfloat16)
        q_ref[:, :] = rope(jnp.dot(xvb, wq_ref[:, :].astype(jnp.bfloat16),
                                   preferred_element_type=jnp.float32))
        k_ref[:, :] = rope(jnp.dot(xvb, wk_ref[:, :].astype(jnp.bfloat16),
                                   preferred_element_type=jnp.float32))
        v_ref[:, :] = jnp.dot(xvb, wv_ref[:, :].astype(jnp.bfloat16),
                              preferred_element_type=jnp.float32
                              ).astype(jnp.bfloat16)

        def attn_part(base):
            for h in range(HQ):
                sl = slice(h * DH, (h + 1) * DH)
                s = lax.dot_general(
                    q_ref[pl.ds(base, HALF), sl], k_ref[:, sl],
                    (((1,), (1,)), ((), ())),
                    preferred_element_type=jnp.float32) * SCALE
                m = jnp.max(s, axis=1, keepdims=True)
                w = jnp.exp(s - m)
                w = w / jnp.sum(w, axis=1, keepdims=True)
                ctx_ref[:, sl] = jnp.dot(
                    w.astype(jnp.bfloat16), v_ref[:, sl],
                    preferred_element_type=jnp.float32).astype(jnp.bfloat16)
            part_ref[pl.ds(base, HALF), :] = jnp.dot(
                ctx_ref[:, :], wo_ref[:, :].astype(jnp.bfloat16),
                preferred_element_type=jnp.float32)

        rbufs = [rbuf1, rbuf2, rbuf3, rbuf4]
        sbufs = [sbuf1, sbuf2, sbuf3, sbuf4]

        def rs_step(s):
            sbufs[s][:, :] = part_ref[pl.ds(send[s], SIZES[s]), :].astype(
                jnp.bfloat16)
            return pltpu.make_async_remote_copy(
                src_ref=sbufs[s],
                dst_ref=rbufs[s],
                send_sem=rs_send.at[s],
                recv_sem=rs_recv.at[s],
                device_id=(partners[s],),
                device_id_type=pl.DeviceIdType.MESH,
            )

        if _ABLATE == "nocompute":
            part_ref[:, :] = x_ref[:, :]
        elif _ABLATE == "nocomm":
            attn_part(send[0])
            attn_part(keep[0])
            out_ref[:, :] = part_ref[:, :].astype(jnp.bfloat16)
            return

        if _ABLATE != "nocompute":
            attn_part(send[0])
        rdma1 = rs_step(0)
        rdma1.start()
        if _ABLATE != "nocompute":
            attn_part(keep[0])
        rdma1.wait()
        part_ref[pl.ds(keep[0], SIZES[0]), :] = (
            part_ref[pl.ds(keep[0], SIZES[0]), :]
            + rbuf1[:, :].astype(jnp.float32))

        for s in (1, 2, 3):
            rdma = rs_step(s)
            rdma.start()
            rdma.wait()
            part_ref[pl.ds(keep[s], SIZES[s]), :] = (
                part_ref[pl.ds(keep[s], SIZES[s]), :]
                + rbufs[s][:, :].astype(jnp.float32))

        out_ref[pl.ds(keep[3], 64), :] = part_ref[pl.ds(keep[3], 64),
                                                  :].astype(jnp.bfloat16)

        for s in (3, 2, 1, 0):
            rdma = pltpu.make_async_remote_copy(
                src_ref=out_ref.at[pl.ds(keep[s], SIZES[s]), :],
                dst_ref=out_ref.at[pl.ds(keep[s], SIZES[s]), :],
                send_sem=ag_send.at[s],
                recv_sem=ag_recv.at[s],
                device_id=(partners[s],),
                device_id_type=pl.DeviceIdType.MESH,
            )
            rdma.start()
            rdma.wait()

    out = pl.pallas_call(
        body,
        out_shape=jax.ShapeDtypeStruct((SQ, D), jnp.bfloat16),
        in_specs=[pl.BlockSpec(memory_space=pltpu.VMEM)] * 7,
        out_specs=pl.BlockSpec(memory_space=pltpu.VMEM),
        scratch_shapes=[
            pltpu.VMEM((SQ, HD), jnp.float32),
            pltpu.VMEM((SQ, HD), jnp.float32),
            pltpu.VMEM((SQ, HD), jnp.bfloat16),
            pltpu.VMEM((HALF, HD), jnp.bfloat16),
            pltpu.VMEM((SQ, D), jnp.float32),
            pltpu.VMEM((512, D), jnp.bfloat16),
            pltpu.VMEM((256, D), jnp.bfloat16),
            pltpu.VMEM((128, D), jnp.bfloat16),
            pltpu.VMEM((64, D), jnp.bfloat16),
            pltpu.VMEM((512, D), jnp.bfloat16),
            pltpu.VMEM((256, D), jnp.bfloat16),
            pltpu.VMEM((128, D), jnp.bfloat16),
            pltpu.VMEM((64, D), jnp.bfloat16),
            pltpu.SemaphoreType.DMA((4,)),
            pltpu.SemaphoreType.DMA((4,)),
            pltpu.SemaphoreType.DMA((4,)),
            pltpu.SemaphoreType.DMA((4,)),
        ],
        compiler_params=pltpu.CompilerParams(
            collective_id=None if _ABLATE == "nocomm" else 0,
            vmem_limit_bytes=128 * 1024 * 1024,
        ),
    )(x2, Wq, Wk, Wv, Wo, cos, sin)
    return out.astype(jnp.float32).reshape(1, SQ, D)


# device time: 79353 ns/iter; 1.1664x vs baseline; 1.0085x over previous
import os
import numpy as np
import jax
import jax.numpy as jnp
from jax import lax
from jax.experimental import pallas as pl
from jax.experimental.pallas import tpu as pltpu

N_DEV = 16
SQ = 1024
D = 1024
HQ = 8
DH = 128
HD = HQ * DH
HALF = SQ // 2
COLS = D // 2
SCALE = 0.08838834764831843
SIZES = (512, 256, 128, 64)


def _rope_tables():
    inv = 1.0 / (10000.0 ** (np.arange(0, DH, 2) / DH))
    pos = np.arange(SQ)[:, None] * inv[None, :]
    cos = np.repeat(np.cos(pos), 2, axis=-1).astype(np.float32)
    sin = np.repeat(np.sin(pos), 2, axis=-1).astype(np.float32)
    return cos, sin


_COS, _SIN = _rope_tables()
_ABLATE = os.environ.get("KERNEL_ABLATE", "")


def kernel(x, Wq, Wk, Wv, Wo):
    x2 = x.reshape(SQ, D)
    cos = jnp.asarray(_COS)
    sin = jnp.asarray(_SIN)

    def body(x_ref, wq_ref, wk_ref, wv_ref, wo_ref, cos_ref, sin_ref,
             out_ref, q_ref, k_ref, v_ref, ctx_ref, part_ref,
             sA0, sA1, sA2, sA3, sB0, sB1, sB2, sB3,
             rA0, rA1, rA2, rA3, rB0, rB1, rB2, rB3,
             rs_send, rs_recv, ag_send, ag_recv):
        my = lax.axis_index("i")
        z = my // 4
        r = my % 4
        y = jnp.where(r >= 2, 1, 0)
        xc = jnp.where((r == 1) | (r == 2), 1, 0)
        z0 = z % 2
        z1 = z // 2

        def lid(xx, yy, zz):
            return 4 * zz + 3 * yy + xx * (1 - 2 * yy)

        p_y = lid(xc, 1 - y, z)
        p_x = lid(1 - xc, y, z)
        p_z0 = lid(xc, y, z + 1 - 2 * z0)
        p_z1 = lid(xc, y, z + 2 - 4 * z1)

        def offsets(bits):
            keep, send = [], []
            base = 0
            for i, b in enumerate(bits):
                sz = SIZES[i]
                keep.append(base + b * sz)
                send.append(base + (1 - b) * sz)
                base = keep[i]
            return keep, send

        keepA, sendA = offsets([y, xc, z0, z1])
        keepB, sendB = offsets([z0, y, xc, z1])
        trees = [
            dict(ti=0, c0=0, partners=[p_y, p_x, p_z0, p_z1],
                 keep=keepA, send=sendA,
                 sbufs=[sA0, sA1, sA2, sA3], rbufs=[rA0, rA1, rA2, rA3]),
            dict(ti=1, c0=COLS, partners=[p_z0, p_y, p_x, p_z1],
                 keep=keepB, send=sendB,
                 sbufs=[sB0, sB1, sB2, sB3], rbufs=[rB0, rB1, rB2, rB3]),
        ]

        if _ABLATE != "nocomm":
            barrier = pltpu.get_barrier_semaphore()
            for p in (p_y, p_x, p_z0, p_z1):
                pl.semaphore_signal(barrier, inc=1, device_id=(p,),
                                    device_id_type=pl.DeviceIdType.MESH)
            pl.semaphore_wait(barrier, 4)

        xv = x_ref[:, :]
        cosv = jnp.concatenate([cos_ref[:, :]] * HQ, axis=1)
        sinv = jnp.concatenate([sin_ref[:, :]] * HQ, axis=1)
        col = lax.broadcasted_iota(jnp.int32, (SQ, HD), 1)
        even = (col % 2) == 0

        def rope(t):
            t_next = pltpu.roll(t, HD - 1, 1)
            t_prev = pltpu.roll(t, 1, 1)
            t_r = jnp.where(even, -t_next, t_prev)
            return t * cosv + t_r * sinv

        if _ABLATE != "nocompute":
            xvb = xv.astype(jnp.bfloat16)
            q_ref[:, :] = rope(jnp.dot(
                xvb, wq_ref[:, :].astype(jnp.bfloat16),
                preferred_element_type=jnp.float32))
            k_ref[:, :] = rope(jnp.dot(
                xvb, wk_ref[:, :].astype(jnp.bfloat16),
                preferred_element_type=jnp.float32))
            v_ref[:, :] = jnp.dot(
                xvb, wv_ref[:, :].astype(jnp.bfloat16),
                preferred_element_type=jnp.float32).astype(jnp.bfloat16)

        def attn_part(base):
            for h in range(HQ):
                sl = slice(h * DH, (h + 1) * DH)
                s = lax.dot_general(
                    q_ref[pl.ds(base, HALF), sl], k_ref[:, sl],
                    (((1,), (1,)), ((), ())),
                    preferred_element_type=jnp.float32) * SCALE
                m = jnp.max(s, axis=1, keepdims=True)
                w = jnp.exp(s - m)
                w = w / jnp.sum(w, axis=1, keepdims=True)
                ctx_ref[:, sl] = jnp.dot(
                    w.astype(jnp.bfloat16), v_ref[:, sl],
                    preferred_element_type=jnp.float32).astype(jnp.bfloat16)
            part_ref[pl.ds(base, HALF), :] = jnp.dot(
                ctx_ref[:, :], wo_ref[:, :].astype(jnp.bfloat16),
                preferred_element_type=jnp.float32)

        def rs_start(t, s):
            t["sbufs"][s][:, :] = part_ref[
                pl.ds(t["send"][s], SIZES[s]),
                t["c0"]:t["c0"] + COLS].astype(jnp.bfloat16)
            rdma = pltpu.make_async_remote_copy(
                src_ref=t["sbufs"][s],
                dst_ref=t["rbufs"][s],
                send_sem=rs_send.at[t["ti"], s],
                recv_sem=rs_recv.at[t["ti"], s],
                device_id=(t["partners"][s],),
                device_id_type=pl.DeviceIdType.MESH,
            )
            rdma.start()
            return rdma

        def rs_finish(t, s, rdma):
            rdma.wait()
            part_ref[pl.ds(t["keep"][s], SIZES[s]),
                     t["c0"]:t["c0"] + COLS] = (
                part_ref[pl.ds(t["keep"][s], SIZES[s]),
                         t["c0"]:t["c0"] + COLS]
                + t["rbufs"][s][:, :].astype(jnp.float32))

        def ag_start(t, s):
            rdma = pltpu.make_async_remote_copy(
                src_ref=out_ref.at[pl.ds(t["keep"][s], SIZES[s]),
                                   pl.ds(t["c0"], COLS)],
                dst_ref=out_ref.at[pl.ds(t["keep"][s], SIZES[s]),
                                   pl.ds(t["c0"], COLS)],
                send_sem=ag_send.at[t["ti"], s],
                recv_sem=ag_recv.at[t["ti"], s],
                device_id=(t["partners"][s],),
                device_id_type=pl.DeviceIdType.MESH,
            )
            rdma.start()
            return rdma

        A, B = trees

        if _ABLATE == "nocompute":
            part_ref[:, :] = x_ref[:, :]
        elif _ABLATE == "nocomm":
            attn_part(sendA[0])
            attn_part(keepA[0])
            out_ref[:, :] = part_ref[:, :].astype(jnp.bfloat16)
            return

        if _ABLATE != "nocompute":
            attn_part(sendA[0])
        a = rs_start(A, 0)
        if _ABLATE != "nocompute":
            attn_part(keepA[0])
        b = rs_start(B, 0)
        for s in (1, 2, 3):
            rs_finish(A, s - 1, a)
            a = rs_start(A, s)
            rs_finish(B, s - 1, b)
            b = rs_start(B, s)
        rs_finish(A, 3, a)
        out_ref[pl.ds(keepA[3], 64), pl.ds(0, COLS)] = part_ref[
            pl.ds(keepA[3], 64), 0:COLS].astype(jnp.bfloat16)
        ag_a = ag_start(A, 3)
        rs_finish(B, 3, b)
        out_ref[pl.ds(keepB[3], 64), pl.ds(COLS, COLS)] = part_ref[
            pl.ds(keepB[3], 64), COLS:D].astype(jnp.bfloat16)
        ag_b = ag_start(B, 3)

        for s in (2, 1, 0):
            ag_a.wait()
            ag_a = ag_start(A, s)
            ag_b.wait()
            ag_b = ag_start(B, s)
        ag_a.wait()
        ag_b.wait()

    out = pl.pallas_call(
        body,
        out_shape=jax.ShapeDtypeStruct((SQ, D), jnp.bfloat16),
        in_specs=[pl.BlockSpec(memory_space=pltpu.VMEM)] * 7,
        out_specs=pl.BlockSpec(memory_space=pltpu.VMEM),
        scratch_shapes=[
            pltpu.VMEM((SQ, HD), jnp.float32),
            pltpu.VMEM((SQ, HD), jnp.float32),
            pltpu.VMEM((SQ, HD), jnp.bfloat16),
            pltpu.VMEM((HALF, HD), jnp.bfloat16),
            pltpu.VMEM((SQ, D), jnp.float32),
            pltpu.VMEM((512, COLS), jnp.bfloat16),
            pltpu.VMEM((256, COLS), jnp.bfloat16),
            pltpu.VMEM((128, COLS), jnp.bfloat16),
            pltpu.VMEM((64, COLS), jnp.bfloat16),
            pltpu.VMEM((512, COLS), jnp.bfloat16),
            pltpu.VMEM((256, COLS), jnp.bfloat16),
            pltpu.VMEM((128, COLS), jnp.bfloat16),
            pltpu.VMEM((64, COLS), jnp.bfloat16),
            pltpu.VMEM((512, COLS), jnp.bfloat16),
            pltpu.VMEM((256, COLS), jnp.bfloat16),
            pltpu.VMEM((128, COLS), jnp.bfloat16),
            pltpu.VMEM((64, COLS), jnp.bfloat16),
            pltpu.VMEM((512, COLS), jnp.bfloat16),
            pltpu.VMEM((256, COLS), jnp.bfloat16),
            pltpu.VMEM((128, COLS), jnp.bfloat16),
            pltpu.VMEM((64, COLS), jnp.bfloat16),
            pltpu.SemaphoreType.DMA((2, 4)),
            pltpu.SemaphoreType.DMA((2, 4)),
            pltpu.SemaphoreType.DMA((2, 4)),
            pltpu.SemaphoreType.DMA((2, 4)),
        ],
        compiler_params=pltpu.CompilerParams(
            collective_id=None if _ABLATE == "nocomm" else 0,
            vmem_limit_bytes=128 * 1024 * 1024,
        ),
    )(x2, Wq, Wk, Wv, Wo, cos, sin)
    return out.astype(jnp.float32).reshape(1, SQ, D)


# device time: 34850 ns/iter; 2.6559x vs baseline; 2.2770x over previous
import os
import numpy as np
import jax
import jax.numpy as jnp
from jax import lax
from jax.experimental import pallas as pl
from jax.experimental.pallas import tpu as pltpu

N_DEV = 16
SQ = 1024
D = 1024
HQ = 8
DH = 128
HD = HQ * DH
HALF = SQ // 2
COLS = D // 2
SCALE = 0.08838834764831843
SIZES = (512, 256, 128, 64)


def _rope_tables():
    inv = 1.0 / (10000.0 ** (np.arange(0, DH, 2) / DH))
    pos = np.arange(SQ)[:, None] * inv[None, :]
    cos = np.repeat(np.cos(pos), 2, axis=-1).astype(np.float32)
    sin = np.repeat(np.sin(pos), 2, axis=-1).astype(np.float32)
    return cos, sin


_COS, _SIN = _rope_tables()
_ABLATE = os.environ.get("KERNEL_ABLATE", "")


def kernel(x, Wq, Wk, Wv, Wo):
    x2 = x.reshape(SQ, D)
    cos = jnp.asarray(_COS)
    sin = jnp.asarray(_SIN)

    def body(x_ref, wq_ref, wk_ref, wv_ref, wo_ref, cos_ref, sin_ref,
             out_ref, q_ref, k_ref, v_ref, ctx_ref, part_ref,
             sA0, sA1, sA2, sA3, sB0, sB1, sB2, sB3,
             rA0, rA1, rA2, rA3, rB0, rB1, rB2, rB3,
             rs_send, rs_recv, ag_send, ag_recv):
        my = lax.axis_index("i")
        z = my // 4
        r = my % 4
        y = jnp.where(r >= 2, 1, 0)
        xc = jnp.where((r == 1) | (r == 2), 1, 0)
        z0 = z % 2
        z1 = z // 2

        def lid(xx, yy, zz):
            return 4 * zz + 3 * yy + xx * (1 - 2 * yy)

        p_y = lid(xc, 1 - y, z)
        p_x = lid(1 - xc, y, z)
        p_z0 = lid(xc, y, z + 1 - 2 * z0)
        p_z1 = lid(xc, y, z + 2 - 4 * z1)

        def offsets(bits):
            keep, send = [], []
            base = 0
            for i, b in enumerate(bits):
                sz = SIZES[i]
                keep.append(base + b * sz)
                send.append(base + (1 - b) * sz)
                base = keep[i]
            return keep, send

        keepA, sendA = offsets([y, xc, z0, z1])
        keepB, sendB = offsets([z0, y, xc, z1])
        trees = [
            dict(ti=0, c0=0, partners=[p_y, p_x, p_z0, p_z1],
                 keep=keepA, send=sendA,
                 sbufs=[sA0, sA1, sA2, sA3], rbufs=[rA0, rA1, rA2, rA3]),
            dict(ti=1, c0=COLS, partners=[p_z0, p_y, p_x, p_z1],
                 keep=keepB, send=sendB,
                 sbufs=[sB0, sB1, sB2, sB3], rbufs=[rB0, rB1, rB2, rB3]),
        ]

        if _ABLATE != "nocomm":
            barrier = pltpu.get_barrier_semaphore()
            for p in (p_y, p_x, p_z0, p_z1):
                pl.semaphore_signal(barrier, inc=1, device_id=(p,),
                                    device_id_type=pl.DeviceIdType.MESH)
            pl.semaphore_wait(barrier, 4)

        xv = x_ref[:, :]
        cosv = jnp.concatenate([cos_ref[:, :]] * HQ, axis=1)
        sinv = jnp.concatenate([sin_ref[:, :]] * HQ, axis=1)
        col = lax.broadcasted_iota(jnp.int32, (SQ, HD), 1)
        even = (col % 2) == 0

        def rope(t):
            t_next = pltpu.roll(t, HD - 1, 1)
            t_prev = pltpu.roll(t, 1, 1)
            t_r = jnp.where(even, -t_next, t_prev)
            return t * cosv + t_r * sinv

        if _ABLATE != "nocompute":
            xvb = xv.astype(jnp.bfloat16)
            q_ref[:, :] = rope(jnp.dot(
                xvb, wq_ref[:, :].astype(jnp.bfloat16),
                preferred_element_type=jnp.float32)).astype(jnp.bfloat16)
            k_ref[:, :] = rope(jnp.dot(
                xvb, wk_ref[:, :].astype(jnp.bfloat16),
                preferred_element_type=jnp.float32)).astype(jnp.bfloat16)
            v_ref[:, :] = jnp.dot(
                xvb, wv_ref[:, :].astype(jnp.bfloat16),
                preferred_element_type=jnp.float32).astype(jnp.bfloat16)

        def attn_part(base):
            for h in range(HQ):
                sl = slice(h * DH, (h + 1) * DH)
                s = lax.dot_general(
                    q_ref[pl.ds(base, HALF), sl], k_ref[:, sl],
                    (((1,), (1,)), ((), ())),
                    preferred_element_type=jnp.float32) * SCALE
                w = jnp.exp(s)
                denom = jnp.sum(w, axis=1, keepdims=True)
                ctx = jnp.dot(w.astype(jnp.bfloat16), v_ref[:, sl],
                              preferred_element_type=jnp.float32)
                ctx_ref[:, sl] = (ctx / denom).astype(jnp.bfloat16)
            part_ref[pl.ds(base, HALF), :] = jnp.dot(
                ctx_ref[:, :], wo_ref[:, :].astype(jnp.bfloat16),
                preferred_element_type=jnp.float32)

        def rs_start(t, s):
            t["sbufs"][s][:, :] = part_ref[
                pl.ds(t["send"][s], SIZES[s]),
                t["c0"]:t["c0"] + COLS].astype(jnp.bfloat16)
            rdma = pltpu.make_async_remote_copy(
                src_ref=t["sbufs"][s],
                dst_ref=t["rbufs"][s],
                send_sem=rs_send.at[t["ti"], s],
                recv_sem=rs_recv.at[t["ti"], s],
                device_id=(t["partners"][s],),
                device_id_type=pl.DeviceIdType.MESH,
            )
            rdma.start()
            return rdma

        def rs_finish(t, s, rdma):
            rdma.wait()
            part_ref[pl.ds(t["keep"][s], SIZES[s]),
                     t["c0"]:t["c0"] + COLS] = (
                part_ref[pl.ds(t["keep"][s], SIZES[s]),
                         t["c0"]:t["c0"] + COLS]
                + t["rbufs"][s][:, :].astype(jnp.float32))

        def ag_start(t, s):
            rdma = pltpu.make_async_remote_copy(
                src_ref=out_ref.at[pl.ds(t["keep"][s], SIZES[s]),
                                   pl.ds(t["c0"], COLS)],
                dst_ref=out_ref.at[pl.ds(t["keep"][s], SIZES[s]),
                                   pl.ds(t["c0"], COLS)],
                send_sem=ag_send.at[t["ti"], s],
                recv_sem=ag_recv.at[t["ti"], s],
                device_id=(t["partners"][s],),
                device_id_type=pl.DeviceIdType.MESH,
            )
            rdma.start()
            return rdma

        A, B = trees

        if _ABLATE == "nocompute":
            part_ref[:, :] = x_ref[:, :]
        elif _ABLATE == "nocomm":
            attn_part(sendA[0])
            attn_part(keepA[0])
            out_ref[:, :] = part_ref[:, :].astype(jnp.bfloat16)
            return

        if _ABLATE != "nocompute":
            attn_part(sendA[0])
        a = rs_start(A, 0)
        if _ABLATE != "nocompute":
            attn_part(keepA[0])
        b = rs_start(B, 0)
        for s in (1, 2, 3):
            rs_finish(A, s - 1, a)
            a = rs_start(A, s)
            rs_finish(B, s - 1, b)
            b = rs_start(B, s)
        rs_finish(A, 3, a)
        out_ref[pl.ds(keepA[3], 64), pl.ds(0, COLS)] = part_ref[
            pl.ds(keepA[3], 64), 0:COLS].astype(jnp.bfloat16)
        ag_a = ag_start(A, 3)
        rs_finish(B, 3, b)
        out_ref[pl.ds(keepB[3], 64), pl.ds(COLS, COLS)] = part_ref[
            pl.ds(keepB[3], 64), COLS:D].astype(jnp.bfloat16)
        ag_b = ag_start(B, 3)

        for s in (2, 1, 0):
            ag_a.wait()
            ag_a = ag_start(A, s)
            ag_b.wait()
            ag_b = ag_start(B, s)
        ag_a.wait()
        ag_b.wait()

    out = pl.pallas_call(
        body,
        out_shape=jax.ShapeDtypeStruct((SQ, D), jnp.bfloat16),
        in_specs=[pl.BlockSpec(memory_space=pltpu.VMEM)] * 7,
        out_specs=pl.BlockSpec(memory_space=pltpu.VMEM),
        scratch_shapes=[
            pltpu.VMEM((SQ, HD), jnp.bfloat16),
            pltpu.VMEM((SQ, HD), jnp.bfloat16),
            pltpu.VMEM((SQ, HD), jnp.bfloat16),
            pltpu.VMEM((HALF, HD), jnp.bfloat16),
            pltpu.VMEM((SQ, D), jnp.float32),
            pltpu.VMEM((512, COLS), jnp.bfloat16),
            pltpu.VMEM((256, COLS), jnp.bfloat16),
            pltpu.VMEM((128, COLS), jnp.bfloat16),
            pltpu.VMEM((64, COLS), jnp.bfloat16),
            pltpu.VMEM((512, COLS), jnp.bfloat16),
            pltpu.VMEM((256, COLS), jnp.bfloat16),
            pltpu.VMEM((128, COLS), jnp.bfloat16),
            pltpu.VMEM((64, COLS), jnp.bfloat16),
            pltpu.VMEM((512, COLS), jnp.bfloat16),
            pltpu.VMEM((256, COLS), jnp.bfloat16),
            pltpu.VMEM((128, COLS), jnp.bfloat16),
            pltpu.VMEM((64, COLS), jnp.bfloat16),
            pltpu.VMEM((512, COLS), jnp.bfloat16),
            pltpu.VMEM((256, COLS), jnp.bfloat16),
            pltpu.VMEM((128, COLS), jnp.bfloat16),
            pltpu.VMEM((64, COLS), jnp.bfloat16),
            pltpu.SemaphoreType.DMA((2, 4)),
            pltpu.SemaphoreType.DMA((2, 4)),
            pltpu.SemaphoreType.DMA((2, 4)),
            pltpu.SemaphoreType.DMA((2, 4)),
        ],
        compiler_params=pltpu.CompilerParams(
            collective_id=None if _ABLATE == "nocomm" else 0,
            vmem_limit_bytes=128 * 1024 * 1024,
        ),
    )(x2, Wq, Wk, Wv, Wo, cos, sin)
    return out.astype(jnp.float32).reshape(1, SQ, D)
